# baseline (speedup 1.0000x reference)
import functools

import jax
import jax.numpy as jnp
from jax.experimental import pallas as pl
from jax.experimental.pallas import tpu as pltpu


BLK = 64
CHUNK = 8


def _flash_body(counts_ref, order_ref, q_ref, k_ref, v_ref, o_ref, s_ref,
                *, num_blocks, scale):
    i = pl.program_id(1)
    qb = q_ref[0]

    @pl.when((i == 0) | (i == num_blocks - 1))
    def _dense_row():
        s = jax.lax.dot_general(
            qb, k_ref[0], (((1,), (1,)), ((), ())),
            preferred_element_type=jnp.float32) * scale
        m = jnp.max(s, axis=1, keepdims=True)
        p = jnp.exp(s - m)
        l = jnp.sum(p, axis=1, keepdims=True)
        ctx = jax.lax.dot_general(
            p, v_ref[0], (((1,), (0,)), ((), ())),
            preferred_element_type=jnp.float32)
        o_ref[0] = ctx / l

    @pl.when((i != 0) & (i != num_blocks - 1))
    def _sparse_row():
        cnt = counts_ref[i]
        idxs = [order_ref[i, j] for j in range(CHUNK)]
        for j in range(CHUNK):
            kb = k_ref[0, pl.ds(idxs[j] * BLK, BLK), :]
            s_ref[:, pl.ds(j * BLK, BLK)] = jax.lax.dot_general(
                qb, kb, (((1,), (1,)), ((), ())),
                preferred_element_type=jnp.float32)
        s = s_ref[...] * scale
        col = jax.lax.broadcasted_iota(jnp.int32, (BLK, CHUNK * BLK), 1)
        s = jnp.where(col // BLK < cnt, s, -1e30)
        m = jnp.max(s, axis=1, keepdims=True)
        p = jnp.exp(s - m)
        l = jnp.sum(p, axis=1, keepdims=True)
        acc = jnp.zeros((BLK, BLK), jnp.float32)
        for j in range(CHUNK):
            vb = v_ref[0, pl.ds(idxs[j] * BLK, BLK), :]
            acc = acc + jax.lax.dot_general(
                p[:, j * BLK:(j + 1) * BLK], vb, (((1,), (0,)), ((), ())),
                preferred_element_type=jnp.float32)
        o_ref[0] = acc / l


def kernel(query_layer, key_layer, value_layer, attention_mask):
    b, h, s, d = query_layer.shape
    bh = b * h
    nb = s // BLK

    bm = attention_mask[::BLK, ::BLK]
    counts = jnp.sum(bm, axis=1).astype(jnp.int32)
    order = jnp.argsort(-bm, axis=1, stable=True).astype(jnp.int32)

    q = query_layer.reshape(bh, s, d)
    k = key_layer.reshape(bh, s, d)
    v = value_layer.reshape(bh, s, d)

    grid = (bh, nb)
    out = pl.pallas_call(
        functools.partial(_flash_body, num_blocks=nb,
                          scale=1.0 / (d ** 0.5)),
        grid_spec=pltpu.PrefetchScalarGridSpec(
            num_scalar_prefetch=2,
            grid=grid,
            in_specs=[
                pl.BlockSpec((1, BLK, d), lambda g, i, *_: (g, i, 0)),
                pl.BlockSpec((1, s, d), lambda g, i, *_: (g, 0, 0)),
                pl.BlockSpec((1, s, d), lambda g, i, *_: (g, 0, 0)),
            ],
            out_specs=pl.BlockSpec((1, BLK, d), lambda g, i, *_: (g, i, 0)),
            scratch_shapes=[
                pltpu.VMEM((BLK, CHUNK * BLK), jnp.float32),
            ],
        ),
        out_shape=jax.ShapeDtypeStruct((bh, s, d), jnp.float32),
    )(counts, order, q, k, v)
    return out.reshape(b, h, s, d)
